# Initial kernel scaffold; baseline (speedup 1.0000x reference)
import jax
import jax.numpy as jnp
from jax.experimental import pallas as pl


def kernel(x, edge_index, edge_attr, batch, W1, root1, b1, W2, root2, b2, W3, root3, b3, Wl, bl):
    raise NotImplementedError("write your pallas kernel here")


# speedup vs baseline: 3.6170x; 3.6170x over previous
import functools

import jax
import jax.numpy as jnp
from jax import lax
from jax.experimental import pallas as pl
from jax.experimental.pallas import tpu as pltpu
from jax.experimental.pallas import tpu_sc as plsc

N = 10000
E = 320000
F = 128
HF = 64
NB = 64
RBLK = 512
NP = 10240
NGRID = NP // RBLK
NCORES = 2
NSUB = 16
K = 80
ROWS_PER_TILE = 2 * NP // NSUB
WB = 80

def _mesh():
    return plsc.VectorSubcoreMesh(
        core_axis_name="c", subcore_axis_name="s",
        num_cores=NCORES, num_subcores=NSUB)


def _zero16():
    return jnp.zeros((16,), jnp.float32)


def _agg_body(src_hbm, dst_hbm, et_hbm, h_hbm, out_hbm,
              src_v, dst_v, et_v, gi_v, si_v, rows_v, acc_sh, sem):
    c = lax.axis_index("c")
    s = lax.axis_index("s")
    row0 = s * ROWS_PER_TILE

    def zrow(i, carry):
        for j in range(HF // 16):
            rows_v[i, pl.ds(j * 16, 16)] = _zero16()
        return carry
    lax.fori_loop(0, WB, zrow, 0)

    def zcp(w, carry):
        pltpu.sync_copy(rows_v, acc_sh.at[pl.ds(row0 + w * WB, WB), :])
        return carry
    lax.fori_loop(0, ROWS_PER_TILE // WB, zcp, 0)
    plsc.subcore_barrier()

    t_edges = E // NSUB
    base0 = s * t_edges
    goff = c * NP

    def chunk(i, carry):
        b = base0 + i * K
        pltpu.sync_copy(src_hbm.at[pl.ds(b, K)], src_v)
        pltpu.sync_copy(dst_hbm.at[pl.ds(b, K)], dst_v)
        pltpu.sync_copy(et_hbm.at[pl.ds(b, K)], et_v)
        for j in range(K // 16):
            sl = pl.ds(j * 16, 16)
            gi_v[sl] = src_v[sl] + goff
            si_v[sl] = dst_v[sl] + et_v[sl] * NP
        pltpu.async_copy(h_hbm.at[gi_v], rows_v, sem).wait()
        pltpu.sync_copy(rows_v, acc_sh.at[si_v], add=True)
        return carry
    lax.fori_loop(0, t_edges // K, chunk, 0)
    plsc.subcore_barrier()

    def wb(w, carry):
        r = row0 + w * WB
        pltpu.sync_copy(acc_sh.at[pl.ds(r, WB), :], rows_v)
        pltpu.sync_copy(rows_v, out_hbm.at[c, pl.ds(r, WB), :])
        return carry
    lax.fori_loop(0, ROWS_PER_TILE // WB, wb, 0)


def _counts_body(dst_hbm, et_hbm, out_hbm,
                 dst_v, et_v, si_v, ones_v, stage_v, acc_sh, sem):
    c = lax.axis_index("c")
    s = lax.axis_index("s")
    row0 = s * ROWS_PER_TILE

    def zrow(i, carry):
        stage_v[i, :] = _zero16()
        return carry
    lax.fori_loop(0, WB, zrow, 0)

    def zcp(w, carry):
        pltpu.sync_copy(stage_v, acc_sh.at[pl.ds(row0 + w * WB, WB), :])
        return carry
    lax.fori_loop(0, ROWS_PER_TILE // WB, zcp, 0)

    e0 = jnp.where(lax.iota(jnp.int32, 16) == 0,
                   jnp.float32(1.0), jnp.float32(0.0))

    def orow(i, carry):
        ones_v[i, :] = e0
        return carry
    lax.fori_loop(0, K, orow, 0)
    plsc.subcore_barrier()

    t_edges = E // (NCORES * NSUB)
    base0 = c * (E // NCORES) + s * t_edges

    def chunk(i, carry):
        b = base0 + i * K
        pltpu.sync_copy(dst_hbm.at[pl.ds(b, K)], dst_v)
        pltpu.sync_copy(et_hbm.at[pl.ds(b, K)], et_v)
        for j in range(K // 16):
            sl = pl.ds(j * 16, 16)
            si_v[sl] = dst_v[sl] + et_v[sl] * NP
        pltpu.sync_copy(ones_v, acc_sh.at[si_v], add=True)
        return carry
    lax.fori_loop(0, t_edges // K, chunk, 0)
    plsc.subcore_barrier()

    def wbf(w, carry):
        r = row0 + w * WB
        pltpu.sync_copy(acc_sh.at[pl.ds(r, WB), :], stage_v)
        pltpu.sync_copy(stage_v, out_hbm.at[c, pl.ds(r, WB), :])
        return carry
    lax.fori_loop(0, ROWS_PER_TILE // WB, wbf, 0)


def _sc_agg(src, dst, et, h_flat):
    return pl.kernel(
        _agg_body,
        out_type=jax.ShapeDtypeStruct((NCORES, 2 * NP, HF), jnp.float32),
        mesh=_mesh(),
        compiler_params=pltpu.CompilerParams(use_tc_tiling_on_sc=False),
        scratch_types=[
            pltpu.VMEM((K,), jnp.int32),
            pltpu.VMEM((K,), jnp.int32),
            pltpu.VMEM((K,), jnp.int32),
            pltpu.VMEM((K,), jnp.int32),
            pltpu.VMEM((K,), jnp.int32),
            pltpu.VMEM((WB, HF), jnp.float32),
            pltpu.VMEM_SHARED((2 * NP, HF), jnp.float32),
            pltpu.SemaphoreType.DMA,
        ],
    )(src, dst, et, h_flat)


def _sc_counts(dst, et):
    return pl.kernel(
        _counts_body,
        out_type=jax.ShapeDtypeStruct((NCORES, 2 * NP, 16), jnp.float32),
        mesh=_mesh(),
        compiler_params=pltpu.CompilerParams(use_tc_tiling_on_sc=False),
        scratch_types=[
            pltpu.VMEM((K,), jnp.int32),
            pltpu.VMEM((K,), jnp.int32),
            pltpu.VMEM((K,), jnp.int32),
            pltpu.VMEM((K, 16), jnp.float32),
            pltpu.VMEM((WB, 16), jnp.float32),
            pltpu.VMEM_SHARED((2 * NP, 16), jnp.float32),
            pltpu.SemaphoreType.DMA,
        ],
    )(dst, et)


def _layer_tc_body(h_ref, a_ref, c0_ref, c1_ref, root_ref, w_ref, b_ref,
                   o_ref):
    h = jnp.concatenate([h_ref[0], h_ref[1]], axis=1)
    acc = jnp.dot(h, root_ref[...],
                  preferred_element_type=jnp.float32) + b_ref[...]
    cnt = c0_ref[...] + c1_ref[...]
    for r in range(2):
        a = jnp.concatenate([a_ref[0, r], a_ref[1, r]], axis=1)
        inv = 1.0 / jnp.maximum(cnt[r, :, 0:1], 1.0)
        acc = acc + jnp.dot(a * inv, w_ref[r],
                            preferred_element_type=jnp.float32)
    out = jnp.maximum(acc, 0.0)
    o_ref[0] = out[:, :HF]
    o_ref[1] = out[:, HF:]


def _tc_layer(h2, agg4, c0, c1, root, w, b2):
    return pl.pallas_call(
        _layer_tc_body,
        grid=(NGRID,),
        in_specs=[
            pl.BlockSpec((2, RBLK, HF), lambda i: (0, i, 0)),
            pl.BlockSpec((2, 2, RBLK, HF), lambda i: (0, 0, i, 0)),
            pl.BlockSpec((2, RBLK, 16), lambda i: (0, i, 0)),
            pl.BlockSpec((2, RBLK, 16), lambda i: (0, i, 0)),
            pl.BlockSpec((F, F), lambda i: (0, 0)),
            pl.BlockSpec((2, F, F), lambda i: (0, 0, 0)),
            pl.BlockSpec((1, F), lambda i: (0, 0)),
        ],
        out_specs=pl.BlockSpec((2, RBLK, HF), lambda i: (0, i, 0)),
        out_shape=jax.ShapeDtypeStruct((2, NP, HF), jnp.float32),
    )(h2, agg4, c0, c1, root, w, b2)


def _pool_tc_body(h_ref, b3_ref, wl_ref, bl_ref, o_ref, s_acc, c_acc):
    i = pl.program_id(0)

    @pl.when(i == 0)
    def _():
        s_acc[...] = jnp.zeros_like(s_acc)
        c_acc[...] = jnp.zeros_like(c_acc)

    h = jnp.concatenate([h_ref[0], h_ref[1]], axis=1)
    bids = b3_ref[0]
    gids = lax.broadcasted_iota(jnp.int32, (NB, RBLK), 0)
    m = (gids == bids).astype(jnp.float32)
    s_acc[...] += jnp.dot(m, h, preferred_element_type=jnp.float32)
    c_acc[...] += jnp.sum(m, axis=1, keepdims=True)

    @pl.when(i == pl.num_programs(0) - 1)
    def _():
        g = s_acc[...] / jnp.maximum(c_acc[...], 1.0)
        o_ref[...] = jnp.dot(g, wl_ref[...],
                             preferred_element_type=jnp.float32) + bl_ref[...]


def _tc_pool(h2, batch3, wl_pad, bl_pad):
    return pl.pallas_call(
        _pool_tc_body,
        grid=(NGRID,),
        in_specs=[
            pl.BlockSpec((2, RBLK, HF), lambda i: (0, i, 0)),
            pl.BlockSpec((1, 1, RBLK), lambda i: (i, 0, 0)),
            pl.BlockSpec((F, F), lambda i: (0, 0)),
            pl.BlockSpec((1, F), lambda i: (0, 0)),
        ],
        out_specs=pl.BlockSpec((NB, F), lambda i: (0, 0)),
        out_shape=jax.ShapeDtypeStruct((NB, F), jnp.float32),
        scratch_shapes=[
            pltpu.VMEM((NB, F), jnp.float32),
            pltpu.VMEM((NB, F), jnp.float32),
        ],
    )(h2, batch3, wl_pad, bl_pad)


def kernel(x, edge_index, edge_attr, batch,
           W1, root1, b1, W2, root2, b2, W3, root3, b3, Wl, bl):
    src = edge_index[0].astype(jnp.int32)
    dst = edge_index[1].astype(jnp.int32)
    et = edge_attr.astype(jnp.int32)

    x_pad = jnp.zeros((NP, F), jnp.float32).at[:N].set(x)
    h_flat = jnp.concatenate([x_pad[:, :HF], x_pad[:, HF:]], axis=0)

    batch_p = jnp.concatenate(
        [batch.astype(jnp.int32), jnp.full((NP - N,), NB, jnp.int32)]
    ).reshape(NGRID, 1, RBLK)

    counts = _sc_counts(dst, et)
    c0 = counts[0].reshape(2, NP, 16)
    c1 = counts[1].reshape(2, NP, 16)

    h2 = None
    for w, root, b in ((W1, root1, b1), (W2, root2, b2), (W3, root3, b3)):
        agg = _sc_agg(src, dst, et, h_flat)
        h2 = _tc_layer(h_flat.reshape(2, NP, HF),
                       agg.reshape(2, 2, NP, HF),
                       c0, c1, root, w, b.reshape(1, F))
        h_flat = h2.reshape(2 * NP, HF)

    wl_pad = jnp.zeros((F, F), jnp.float32).at[:, :Wl.shape[1]].set(Wl)
    bl_pad = jnp.zeros((1, F), jnp.float32).at[0, :bl.shape[0]].set(bl)
    out = _tc_pool(h2, batch_p, wl_pad, bl_pad)
    return out[:, :Wl.shape[1]]


# speedup vs baseline: 7.4293x; 2.0540x over previous
import functools

import jax
import jax.numpy as jnp
from jax import lax
from jax.experimental import pallas as pl
from jax.experimental.pallas import tpu as pltpu
from jax.experimental.pallas import tpu_sc as plsc

N = 10000
E = 320000
F = 128
HF = 64
NB = 64
RBLK = 512
NP = 10240
NGRID = NP // RBLK
NCORES = 2
NSUB = 16
K = 80
ROWS_PER_TILE = 2 * NP // NSUB
WB = 80

def _mesh():
    return plsc.VectorSubcoreMesh(
        core_axis_name="c", subcore_axis_name="s",
        num_cores=NCORES, num_subcores=NSUB)


def _zero16():
    return jnp.zeros((16,), jnp.float32)


def _agg_body(src_hbm, dst_hbm, et_hbm, h_hbm, out_hbm,
              src_v, dst_v, et_v, gi_v, si_v, rows_v, zero_v, acc_sh,
              isem0, isem1, gsem0, gsem1, ssem0, ssem1):
    c = lax.axis_index("c")
    s = lax.axis_index("s")
    row0 = s * ROWS_PER_TILE

    def zrow(i, carry):
        for j in range(HF // 16):
            zero_v[i, pl.ds(j * 16, 16)] = _zero16()
        return carry
    lax.fori_loop(0, WB, zrow, 0)

    def zcp(w, carry):
        pltpu.sync_copy(zero_v, acc_sh.at[pl.ds(row0 + w * WB, WB), :])
        return carry
    lax.fori_loop(0, ROWS_PER_TILE // WB, zcp, 0)
    plsc.subcore_barrier()

    t_edges = E // NSUB
    base0 = s * t_edges
    goff = c * NP
    nch = t_edges // K

    def issue_idx(i, slot, isem):
        b = base0 + i * K
        pltpu.async_copy(src_hbm.at[pl.ds(b, K)], src_v.at[slot], isem)
        pltpu.async_copy(dst_hbm.at[pl.ds(b, K)], dst_v.at[slot], isem)
        pltpu.async_copy(et_hbm.at[pl.ds(b, K)], et_v.at[slot], isem)

    def wait_idx(slot, isem):
        for _ in range(3):
            pltpu.make_async_copy(
                src_hbm.at[pl.ds(0, K)], src_v.at[slot], isem).wait()

    def compute_idx(slot):
        for j in range(K // 16):
            sl = pl.ds(j * 16, 16)
            gi_v[slot, sl] = src_v[slot, sl] + goff
            si_v[slot, sl] = dst_v[slot, sl] + et_v[slot, sl] * NP

    def wait_rows_bytes(slot, sem):
        pltpu.make_async_copy(
            h_hbm.at[pl.ds(0, K)], rows_v.at[slot], sem).wait()

    def chunk(m, i, slot, isem, gsem, ssem):
        @pl.when(m > 0)
        def _():
            wait_rows_bytes(slot, ssem)
        wait_idx(slot, isem)
        compute_idx(slot)

        @pl.when(m < NHALF - 1)
        def _():
            issue_idx(i + 2, slot, isem)
        pltpu.async_copy(h_hbm.at[gi_v.at[slot]], rows_v.at[slot], gsem)
        wait_rows_bytes(slot, gsem)
        pltpu.async_copy(rows_v.at[slot], acc_sh.at[si_v.at[slot]], ssem,
                         add=True)

    NHALF = nch // 2
    issue_idx(0, 0, isem0)
    issue_idx(1, 1, isem1)

    def step(m, carry):
        chunk(m, 2 * m, 0, isem0, gsem0, ssem0)
        chunk(m, 2 * m + 1, 1, isem1, gsem1, ssem1)
        return carry
    lax.fori_loop(0, NHALF, step, 0)
    wait_rows_bytes(0, ssem0)
    wait_rows_bytes(1, ssem1)
    plsc.subcore_barrier()

    def wb(w, carry):
        r = row0 + w * WB
        pltpu.sync_copy(acc_sh.at[pl.ds(r, WB), :],
                        out_hbm.at[c, pl.ds(r, WB), :])
        return carry
    lax.fori_loop(0, ROWS_PER_TILE // WB, wb, 0)


def _counts_body(dst_hbm, et_hbm, out_hbm,
                 dst_v, et_v, si_v, ones_v, stage_v, acc_sh, sem):
    c = lax.axis_index("c")
    s = lax.axis_index("s")
    row0 = s * ROWS_PER_TILE

    def zrow(i, carry):
        stage_v[i, :] = _zero16()
        return carry
    lax.fori_loop(0, WB, zrow, 0)

    def zcp(w, carry):
        pltpu.sync_copy(stage_v, acc_sh.at[pl.ds(row0 + w * WB, WB), :])
        return carry
    lax.fori_loop(0, ROWS_PER_TILE // WB, zcp, 0)

    e0 = jnp.where(lax.iota(jnp.int32, 16) == 0,
                   jnp.float32(1.0), jnp.float32(0.0))

    def orow(i, carry):
        ones_v[i, :] = e0
        return carry
    lax.fori_loop(0, K, orow, 0)
    plsc.subcore_barrier()

    t_edges = E // (NCORES * NSUB)
    base0 = c * (E // NCORES) + s * t_edges

    def chunk(i, carry):
        b = base0 + i * K
        pltpu.sync_copy(dst_hbm.at[pl.ds(b, K)], dst_v)
        pltpu.sync_copy(et_hbm.at[pl.ds(b, K)], et_v)
        for j in range(K // 16):
            sl = pl.ds(j * 16, 16)
            si_v[sl] = dst_v[sl] + et_v[sl] * NP
        pltpu.sync_copy(ones_v, acc_sh.at[si_v], add=True)
        return carry
    lax.fori_loop(0, t_edges // K, chunk, 0)
    plsc.subcore_barrier()

    def wbf(w, carry):
        r = row0 + w * WB
        pltpu.sync_copy(acc_sh.at[pl.ds(r, WB), :], stage_v)
        pltpu.sync_copy(stage_v, out_hbm.at[c, pl.ds(r, WB), :])
        return carry
    lax.fori_loop(0, ROWS_PER_TILE // WB, wbf, 0)


def _sc_agg(src, dst, et, h_flat):
    return pl.kernel(
        _agg_body,
        out_type=jax.ShapeDtypeStruct((NCORES, 2 * NP, HF), jnp.float32),
        mesh=_mesh(),
        compiler_params=pltpu.CompilerParams(use_tc_tiling_on_sc=False),
        scratch_types=[
            pltpu.VMEM((2, K), jnp.int32),
            pltpu.VMEM((2, K), jnp.int32),
            pltpu.VMEM((2, K), jnp.int32),
            pltpu.VMEM((2, K), jnp.int32),
            pltpu.VMEM((2, K), jnp.int32),
            pltpu.VMEM((2, K, HF), jnp.float32),
            pltpu.VMEM((WB, HF), jnp.float32),
            pltpu.VMEM_SHARED((2 * NP, HF), jnp.float32),
            pltpu.SemaphoreType.DMA,
            pltpu.SemaphoreType.DMA,
            pltpu.SemaphoreType.DMA,
            pltpu.SemaphoreType.DMA,
            pltpu.SemaphoreType.DMA,
            pltpu.SemaphoreType.DMA,
        ],
    )(src, dst, et, h_flat)


def _sc_counts(dst, et):
    return pl.kernel(
        _counts_body,
        out_type=jax.ShapeDtypeStruct((NCORES, 2 * NP, 16), jnp.float32),
        mesh=_mesh(),
        compiler_params=pltpu.CompilerParams(use_tc_tiling_on_sc=False),
        scratch_types=[
            pltpu.VMEM((K,), jnp.int32),
            pltpu.VMEM((K,), jnp.int32),
            pltpu.VMEM((K,), jnp.int32),
            pltpu.VMEM((K, 16), jnp.float32),
            pltpu.VMEM((WB, 16), jnp.float32),
            pltpu.VMEM_SHARED((2 * NP, 16), jnp.float32),
            pltpu.SemaphoreType.DMA,
        ],
    )(dst, et)


def _layer_tc_body(h_ref, a_ref, c0_ref, c1_ref, root_ref, w_ref, b_ref,
                   o_ref):
    h = jnp.concatenate([h_ref[0], h_ref[1]], axis=1)
    acc = jnp.dot(h, root_ref[...],
                  preferred_element_type=jnp.float32) + b_ref[...]
    cnt = c0_ref[...] + c1_ref[...]
    for r in range(2):
        a = jnp.concatenate([a_ref[0, r], a_ref[1, r]], axis=1)
        inv = 1.0 / jnp.maximum(cnt[r, :, 0:1], 1.0)
        acc = acc + jnp.dot(a * inv, w_ref[r],
                            preferred_element_type=jnp.float32)
    out = jnp.maximum(acc, 0.0)
    o_ref[0] = out[:, :HF]
    o_ref[1] = out[:, HF:]


def _tc_layer(h2, agg4, c0, c1, root, w, b2):
    return pl.pallas_call(
        _layer_tc_body,
        grid=(NGRID,),
        in_specs=[
            pl.BlockSpec((2, RBLK, HF), lambda i: (0, i, 0)),
            pl.BlockSpec((2, 2, RBLK, HF), lambda i: (0, 0, i, 0)),
            pl.BlockSpec((2, RBLK, 16), lambda i: (0, i, 0)),
            pl.BlockSpec((2, RBLK, 16), lambda i: (0, i, 0)),
            pl.BlockSpec((F, F), lambda i: (0, 0)),
            pl.BlockSpec((2, F, F), lambda i: (0, 0, 0)),
            pl.BlockSpec((1, F), lambda i: (0, 0)),
        ],
        out_specs=pl.BlockSpec((2, RBLK, HF), lambda i: (0, i, 0)),
        out_shape=jax.ShapeDtypeStruct((2, NP, HF), jnp.float32),
    )(h2, agg4, c0, c1, root, w, b2)


def _pool_tc_body(h_ref, b3_ref, wl_ref, bl_ref, o_ref, s_acc, c_acc):
    i = pl.program_id(0)

    @pl.when(i == 0)
    def _():
        s_acc[...] = jnp.zeros_like(s_acc)
        c_acc[...] = jnp.zeros_like(c_acc)

    h = jnp.concatenate([h_ref[0], h_ref[1]], axis=1)
    bids = b3_ref[0]
    gids = lax.broadcasted_iota(jnp.int32, (NB, RBLK), 0)
    m = (gids == bids).astype(jnp.float32)
    s_acc[...] += jnp.dot(m, h, preferred_element_type=jnp.float32)
    c_acc[...] += jnp.sum(m, axis=1, keepdims=True)

    @pl.when(i == pl.num_programs(0) - 1)
    def _():
        g = s_acc[...] / jnp.maximum(c_acc[...], 1.0)
        o_ref[...] = jnp.dot(g, wl_ref[...],
                             preferred_element_type=jnp.float32) + bl_ref[...]


def _tc_pool(h2, batch3, wl_pad, bl_pad):
    return pl.pallas_call(
        _pool_tc_body,
        grid=(NGRID,),
        in_specs=[
            pl.BlockSpec((2, RBLK, HF), lambda i: (0, i, 0)),
            pl.BlockSpec((1, 1, RBLK), lambda i: (i, 0, 0)),
            pl.BlockSpec((F, F), lambda i: (0, 0)),
            pl.BlockSpec((1, F), lambda i: (0, 0)),
        ],
        out_specs=pl.BlockSpec((NB, F), lambda i: (0, 0)),
        out_shape=jax.ShapeDtypeStruct((NB, F), jnp.float32),
        scratch_shapes=[
            pltpu.VMEM((NB, F), jnp.float32),
            pltpu.VMEM((NB, F), jnp.float32),
        ],
    )(h2, batch3, wl_pad, bl_pad)


def kernel(x, edge_index, edge_attr, batch,
           W1, root1, b1, W2, root2, b2, W3, root3, b3, Wl, bl):
    src = edge_index[0].astype(jnp.int32)
    dst = edge_index[1].astype(jnp.int32)
    et = edge_attr.astype(jnp.int32)

    x_pad = jnp.zeros((NP, F), jnp.float32).at[:N].set(x)
    h_flat = jnp.concatenate([x_pad[:, :HF], x_pad[:, HF:]], axis=0)

    batch_p = jnp.concatenate(
        [batch.astype(jnp.int32), jnp.full((NP - N,), NB, jnp.int32)]
    ).reshape(NGRID, 1, RBLK)

    counts = _sc_counts(dst, et)
    c0 = counts[0].reshape(2, NP, 16)
    c1 = counts[1].reshape(2, NP, 16)

    h2 = None
    for w, root, b in ((W1, root1, b1), (W2, root2, b2), (W3, root3, b3)):
        agg = _sc_agg(src, dst, et, h_flat)
        h2 = _tc_layer(h_flat.reshape(2, NP, HF),
                       agg.reshape(2, 2, NP, HF),
                       c0, c1, root, w, b.reshape(1, F))
        h_flat = h2.reshape(2 * NP, HF)

    wl_pad = jnp.zeros((F, F), jnp.float32).at[:, :Wl.shape[1]].set(Wl)
    bl_pad = jnp.zeros((1, F), jnp.float32).at[0, :bl.shape[0]].set(bl)
    out = _tc_pool(h2, batch_p, wl_pad, bl_pad)
    return out[:, :Wl.shape[1]]


# speedup vs baseline: 11.5051x; 1.5486x over previous
import functools

import jax
import jax.numpy as jnp
from jax import lax
from jax.experimental import pallas as pl
from jax.experimental.pallas import tpu as pltpu
from jax.experimental.pallas import tpu_sc as plsc

N = 10000
E = 320000
F = 128
HF = 64
NB = 64
RBLK = 512
NP = 10240
NGRID = NP // RBLK
NCORES = 2
NSUB = 16
K = 80
ROWS_PER_TILE = 2 * NP // NSUB
WB = 80

def _mesh():
    return plsc.VectorSubcoreMesh(
        core_axis_name="c", subcore_axis_name="s",
        num_cores=NCORES, num_subcores=NSUB)


def _zero16():
    return jnp.zeros((16,), jnp.float32)


NSLOT = 4
LAG = 2


def _agg_body(src_hbm, dst_hbm, et_hbm, h_hbm, out_hbm,
              src_v, dst_v, et_v, gi_v, si_v, rows_v, zero_v, acc_sh,
              isem0, isem1, isem2, isem3,
              gsem0, gsem1, gsem2, gsem3,
              ssem0, ssem1, ssem2, ssem3, wsem):
    isems = (isem0, isem1, isem2, isem3)
    gsems = (gsem0, gsem1, gsem2, gsem3)
    ssems = (ssem0, ssem1, ssem2, ssem3)
    c = lax.axis_index("c")
    s = lax.axis_index("s")
    row0 = s * ROWS_PER_TILE

    def zrow(i, carry):
        for j in range(HF // 16):
            zero_v[i, pl.ds(j * 16, 16)] = _zero16()
        return carry
    lax.fori_loop(0, WB, zrow, 0)

    def zcp(w, carry):
        pltpu.sync_copy(zero_v, acc_sh.at[pl.ds(row0 + w * WB, WB), :])
        return carry
    lax.fori_loop(0, ROWS_PER_TILE // WB, zcp, 0)
    plsc.subcore_barrier()

    t_edges = E // NSUB
    base0 = s * t_edges
    goff = c * NP
    nch = t_edges // K

    def issue_idx(i, slot):
        b = base0 + i * K
        pltpu.async_copy(src_hbm.at[pl.ds(b, K)], src_v.at[slot], isems[slot])
        pltpu.async_copy(dst_hbm.at[pl.ds(b, K)], dst_v.at[slot], isems[slot])
        pltpu.async_copy(et_hbm.at[pl.ds(b, K)], et_v.at[slot], isems[slot])

    def wait_idx(slot):
        for _ in range(3):
            pltpu.make_async_copy(
                src_hbm.at[pl.ds(0, K)], src_v.at[slot], isems[slot]).wait()

    def compute_idx(slot):
        for j in range(K // 16):
            sl = pl.ds(j * 16, 16)
            gi_v[slot, sl] = src_v[slot, sl] + goff
            si_v[slot, sl] = dst_v[slot, sl] + et_v[slot, sl] * NP

    def wait_rows_bytes(slot, sem):
        pltpu.make_async_copy(
            h_hbm.at[pl.ds(0, K)], rows_v.at[slot], sem).wait()

    def issue_scatter(slot):
        pltpu.async_copy(rows_v.at[slot], acc_sh.at[si_v.at[slot]],
                         ssems[slot], add=True)

    def chunk(i, slot):
        @pl.when(i >= NSLOT)
        def _():
            wait_rows_bytes(slot, ssems[slot])
        wait_idx(slot)
        compute_idx(slot)

        @pl.when(i + NSLOT < nch)
        def _():
            issue_idx(i + NSLOT, slot)
        pltpu.async_copy(h_hbm.at[gi_v.at[slot]], rows_v.at[slot],
                         gsems[slot])
        q = (slot - LAG) % NSLOT

        @pl.when(i >= LAG)
        def _():
            wait_rows_bytes(q, gsems[q])
            issue_scatter(q)

    for p in range(NSLOT):
        issue_idx(p, p)

    nmain = (nch // NSLOT) * NSLOT

    def step(m, carry):
        for p in range(NSLOT):
            chunk(NSLOT * m + p, p)
        return carry
    lax.fori_loop(0, nch // NSLOT, step, 0)
    for i in range(nmain, nch):
        chunk(jnp.int32(i), i % NSLOT)
    for i in range(nch - LAG, nch):
        q = i % NSLOT
        wait_rows_bytes(q, gsems[q])
        issue_scatter(q)
    for i in range(nch - NSLOT, nch):
        wait_rows_bytes(i % NSLOT, ssems[i % NSLOT])
    plsc.subcore_barrier()

    nwb = ROWS_PER_TILE // WB

    def wb(w, carry):
        r = row0 + w * WB
        pltpu.async_copy(acc_sh.at[pl.ds(r, WB), :],
                         out_hbm.at[c, pl.ds(r, WB), :], wsem)
        return carry
    lax.fori_loop(0, nwb, wb, 0)

    def wbw(w, carry):
        pltpu.make_async_copy(
            acc_sh.at[pl.ds(row0, WB), :],
            out_hbm.at[c, pl.ds(row0, WB), :], wsem).wait()
        return carry
    lax.fori_loop(0, nwb, wbw, 0)


def _counts_body(dst_hbm, et_hbm, out_hbm,
                 dst_v, et_v, si_v, ones_v, stage_v, acc_sh, sem):
    c = lax.axis_index("c")
    s = lax.axis_index("s")
    row0 = s * ROWS_PER_TILE

    def zrow(i, carry):
        stage_v[i, :] = _zero16()
        return carry
    lax.fori_loop(0, WB, zrow, 0)

    def zcp(w, carry):
        pltpu.sync_copy(stage_v, acc_sh.at[pl.ds(row0 + w * WB, WB), :])
        return carry
    lax.fori_loop(0, ROWS_PER_TILE // WB, zcp, 0)

    e0 = jnp.where(lax.iota(jnp.int32, 16) == 0,
                   jnp.float32(1.0), jnp.float32(0.0))

    def orow(i, carry):
        ones_v[i, :] = e0
        return carry
    lax.fori_loop(0, K, orow, 0)
    plsc.subcore_barrier()

    t_edges = E // (NCORES * NSUB)
    base0 = c * (E // NCORES) + s * t_edges

    def chunk(i, carry):
        b = base0 + i * K
        pltpu.sync_copy(dst_hbm.at[pl.ds(b, K)], dst_v)
        pltpu.sync_copy(et_hbm.at[pl.ds(b, K)], et_v)
        for j in range(K // 16):
            sl = pl.ds(j * 16, 16)
            si_v[sl] = dst_v[sl] + et_v[sl] * NP
        pltpu.sync_copy(ones_v, acc_sh.at[si_v], add=True)
        return carry
    lax.fori_loop(0, t_edges // K, chunk, 0)
    plsc.subcore_barrier()

    def wbf(w, carry):
        r = row0 + w * WB
        pltpu.sync_copy(acc_sh.at[pl.ds(r, WB), :], stage_v)
        pltpu.sync_copy(stage_v, out_hbm.at[c, pl.ds(r, WB), :])
        return carry
    lax.fori_loop(0, ROWS_PER_TILE // WB, wbf, 0)


def _sc_agg(src, dst, et, h_flat):
    return pl.kernel(
        _agg_body,
        out_type=jax.ShapeDtypeStruct((NCORES, 2 * NP, HF), jnp.float32),
        mesh=_mesh(),
        compiler_params=pltpu.CompilerParams(use_tc_tiling_on_sc=False),
        scratch_types=[
            pltpu.VMEM((NSLOT, K), jnp.int32),
            pltpu.VMEM((NSLOT, K), jnp.int32),
            pltpu.VMEM((NSLOT, K), jnp.int32),
            pltpu.VMEM((NSLOT, K), jnp.int32),
            pltpu.VMEM((NSLOT, K), jnp.int32),
            pltpu.VMEM((NSLOT, K, HF), jnp.float32),
            pltpu.VMEM((WB, HF), jnp.float32),
            pltpu.VMEM_SHARED((2 * NP, HF), jnp.float32),
        ] + [pltpu.SemaphoreType.DMA] * 13,
    )(src, dst, et, h_flat)


def _sc_counts(dst, et):
    return pl.kernel(
        _counts_body,
        out_type=jax.ShapeDtypeStruct((NCORES, 2 * NP, 16), jnp.float32),
        mesh=_mesh(),
        compiler_params=pltpu.CompilerParams(use_tc_tiling_on_sc=False),
        scratch_types=[
            pltpu.VMEM((K,), jnp.int32),
            pltpu.VMEM((K,), jnp.int32),
            pltpu.VMEM((K,), jnp.int32),
            pltpu.VMEM((K, 16), jnp.float32),
            pltpu.VMEM((WB, 16), jnp.float32),
            pltpu.VMEM_SHARED((2 * NP, 16), jnp.float32),
            pltpu.SemaphoreType.DMA,
        ],
    )(dst, et)


def _layer_tc_body(h_ref, a_ref, c0_ref, c1_ref, root_ref, w_ref, b_ref,
                   o_ref):
    h = jnp.concatenate([h_ref[0], h_ref[1]], axis=1)
    acc = jnp.dot(h, root_ref[...],
                  preferred_element_type=jnp.float32) + b_ref[...]
    cnt = c0_ref[...] + c1_ref[...]
    for r in range(2):
        a = jnp.concatenate([a_ref[0, r], a_ref[1, r]], axis=1)
        inv = 1.0 / jnp.maximum(cnt[r, :, 0:1], 1.0)
        acc = acc + jnp.dot(a * inv, w_ref[r],
                            preferred_element_type=jnp.float32)
    out = jnp.maximum(acc, 0.0)
    o_ref[0] = out[:, :HF]
    o_ref[1] = out[:, HF:]


def _tc_layer(h2, agg4, c0, c1, root, w, b2):
    return pl.pallas_call(
        _layer_tc_body,
        grid=(NGRID,),
        in_specs=[
            pl.BlockSpec((2, RBLK, HF), lambda i: (0, i, 0)),
            pl.BlockSpec((2, 2, RBLK, HF), lambda i: (0, 0, i, 0)),
            pl.BlockSpec((2, RBLK, 16), lambda i: (0, i, 0)),
            pl.BlockSpec((2, RBLK, 16), lambda i: (0, i, 0)),
            pl.BlockSpec((F, F), lambda i: (0, 0)),
            pl.BlockSpec((2, F, F), lambda i: (0, 0, 0)),
            pl.BlockSpec((1, F), lambda i: (0, 0)),
        ],
        out_specs=pl.BlockSpec((2, RBLK, HF), lambda i: (0, i, 0)),
        out_shape=jax.ShapeDtypeStruct((2, NP, HF), jnp.float32),
    )(h2, agg4, c0, c1, root, w, b2)


def _pool_tc_body(h_ref, b3_ref, wl_ref, bl_ref, o_ref, s_acc, c_acc):
    i = pl.program_id(0)

    @pl.when(i == 0)
    def _():
        s_acc[...] = jnp.zeros_like(s_acc)
        c_acc[...] = jnp.zeros_like(c_acc)

    h = jnp.concatenate([h_ref[0], h_ref[1]], axis=1)
    bids = b3_ref[0]
    gids = lax.broadcasted_iota(jnp.int32, (NB, RBLK), 0)
    m = (gids == bids).astype(jnp.float32)
    s_acc[...] += jnp.dot(m, h, preferred_element_type=jnp.float32)
    c_acc[...] += jnp.sum(m, axis=1, keepdims=True)

    @pl.when(i == pl.num_programs(0) - 1)
    def _():
        g = s_acc[...] / jnp.maximum(c_acc[...], 1.0)
        o_ref[...] = jnp.dot(g, wl_ref[...],
                             preferred_element_type=jnp.float32) + bl_ref[...]


def _tc_pool(h2, batch3, wl_pad, bl_pad):
    return pl.pallas_call(
        _pool_tc_body,
        grid=(NGRID,),
        in_specs=[
            pl.BlockSpec((2, RBLK, HF), lambda i: (0, i, 0)),
            pl.BlockSpec((1, 1, RBLK), lambda i: (i, 0, 0)),
            pl.BlockSpec((F, F), lambda i: (0, 0)),
            pl.BlockSpec((1, F), lambda i: (0, 0)),
        ],
        out_specs=pl.BlockSpec((NB, F), lambda i: (0, 0)),
        out_shape=jax.ShapeDtypeStruct((NB, F), jnp.float32),
        scratch_shapes=[
            pltpu.VMEM((NB, F), jnp.float32),
            pltpu.VMEM((NB, F), jnp.float32),
        ],
    )(h2, batch3, wl_pad, bl_pad)


def kernel(x, edge_index, edge_attr, batch,
           W1, root1, b1, W2, root2, b2, W3, root3, b3, Wl, bl):
    src = edge_index[0].astype(jnp.int32)
    dst = edge_index[1].astype(jnp.int32)
    et = edge_attr.astype(jnp.int32)

    x_pad = jnp.zeros((NP, F), jnp.float32).at[:N].set(x)
    h_flat = jnp.concatenate([x_pad[:, :HF], x_pad[:, HF:]], axis=0)

    batch_p = jnp.concatenate(
        [batch.astype(jnp.int32), jnp.full((NP - N,), NB, jnp.int32)]
    ).reshape(NGRID, 1, RBLK)

    counts = _sc_counts(dst, et)
    c0 = counts[0].reshape(2, NP, 16)
    c1 = counts[1].reshape(2, NP, 16)

    h2 = None
    for w, root, b in ((W1, root1, b1), (W2, root2, b2), (W3, root3, b3)):
        agg = _sc_agg(src, dst, et, h_flat)
        h2 = _tc_layer(h_flat.reshape(2, NP, HF),
                       agg.reshape(2, 2, NP, HF),
                       c0, c1, root, w, b.reshape(1, F))
        h_flat = h2.reshape(2 * NP, HF)

    wl_pad = jnp.zeros((F, F), jnp.float32).at[:, :Wl.shape[1]].set(Wl)
    bl_pad = jnp.zeros((1, F), jnp.float32).at[0, :bl.shape[0]].set(bl)
    out = _tc_pool(h2, batch_p, wl_pad, bl_pad)
    return out[:, :Wl.shape[1]]


# speedup vs baseline: 14.4730x; 1.2580x over previous
import jax
import jax.numpy as jnp
from jax import lax
from jax.experimental import pallas as pl
from jax.experimental.pallas import tpu as pltpu
from jax.experimental.pallas import tpu_sc as plsc

N = 10000
E = 320000
F = 128
HF = 64
NB = 64
RBLK = 512
NP = 10240
NGRID = NP // RBLK
NCORES = 2
NSUB = 16
K = 80
ROWS_PER_TILE = 2 * NP // NSUB
WB = 80
NSLOT = 4
LAG = 2


def _mesh():
    return plsc.VectorSubcoreMesh(
        core_axis_name="c", subcore_axis_name="s",
        num_cores=NCORES, num_subcores=NSUB)


def _zero16():
    return jnp.zeros((16,), jnp.float32)


def _build_agg_body(with_counts):
    def body(*args):
        if with_counts:
            (src_hbm, dst_hbm, et_hbm, h_hbm, out_hbm, cnt_hbm,
             src_v, dst_v, et_v, gi_v, si_v, rows_v, zero_v, ones_v,
             acc_sh, acc2_sh, *sems) = args
        else:
            (src_hbm, dst_hbm, et_hbm, h_hbm, out_hbm,
             src_v, dst_v, et_v, gi_v, si_v, rows_v, zero_v,
             acc_sh, *sems) = args
            cnt_hbm = ones_v = acc2_sh = None
        isems = sems[0:4]
        gsems = sems[4:8]
        ssems = sems[8:12]
        if with_counts:
            osems = sems[12:16]
            wsem, wsem2 = sems[16], sems[17]
        else:
            wsem = sems[12]
        c = lax.axis_index("c")
        s = lax.axis_index("s")
        row0 = s * ROWS_PER_TILE

        def zrow(i, carry):
            for j in range(HF // 16):
                zero_v[i, pl.ds(j * 16, 16)] = _zero16()
            return carry
        lax.fori_loop(0, WB, zrow, 0)

        def zcp(w, carry):
            pltpu.sync_copy(zero_v, acc_sh.at[pl.ds(row0 + w * WB, WB), :])
            return carry
        lax.fori_loop(0, ROWS_PER_TILE // WB, zcp, 0)

        if with_counts:
            def z2row(i, carry):
                ones_v[i, :] = _zero16()
                return carry
            lax.fori_loop(0, K, z2row, 0)

            def z2cp(w, carry):
                pltpu.sync_copy(ones_v.at[pl.ds(0, WB), :],
                                acc2_sh.at[pl.ds(row0 + w * WB, WB), :])
                return carry
            lax.fori_loop(0, ROWS_PER_TILE // WB, z2cp, 0)

            e0 = jnp.where(lax.iota(jnp.int32, 16) == 0,
                           jnp.float32(1.0), jnp.float32(0.0))

            def orow(i, carry):
                ones_v[i, :] = e0
                return carry
            lax.fori_loop(0, K, orow, 0)
        plsc.subcore_barrier()

        t_edges = E // NSUB
        base0 = s * t_edges
        goff = c * NP
        nch = t_edges // K

        def issue_idx(i, slot):
            b = base0 + i * K
            pltpu.async_copy(src_hbm.at[pl.ds(b, K)], src_v.at[slot],
                             isems[slot])
            pltpu.async_copy(dst_hbm.at[pl.ds(b, K)], dst_v.at[slot],
                             isems[slot])
            pltpu.async_copy(et_hbm.at[pl.ds(b, K)], et_v.at[slot],
                             isems[slot])

        def wait_idx(slot):
            for _ in range(3):
                pltpu.make_async_copy(
                    src_hbm.at[pl.ds(0, K)], src_v.at[slot],
                    isems[slot]).wait()

        def compute_idx(slot):
            for j in range(K // 16):
                sl = pl.ds(j * 16, 16)
                gi_v[slot, sl] = src_v[slot, sl] + goff
                si_v[slot, sl] = dst_v[slot, sl] + et_v[slot, sl] * NP

        def wait_rows_bytes(slot, sem):
            pltpu.make_async_copy(
                h_hbm.at[pl.ds(0, K)], rows_v.at[slot], sem).wait()

        def issue_scatter(slot):
            pltpu.async_copy(rows_v.at[slot], acc_sh.at[si_v.at[slot]],
                             ssems[slot], add=True)
            if with_counts:
                pltpu.async_copy(ones_v, acc2_sh.at[si_v.at[slot]],
                                 osems[slot], add=True)

        def wait_scatter(slot):
            wait_rows_bytes(slot, ssems[slot])
            if with_counts:
                pltpu.make_async_copy(
                    cnt_hbm.at[0, pl.ds(0, K), :], ones_v,
                    osems[slot]).wait()

        def chunk(i, slot):
            @pl.when(i >= NSLOT)
            def _():
                wait_scatter(slot)
            wait_idx(slot)
            compute_idx(slot)

            @pl.when(i + NSLOT < nch)
            def _():
                issue_idx(i + NSLOT, slot)
            pltpu.async_copy(h_hbm.at[gi_v.at[slot]], rows_v.at[slot],
                             gsems[slot])
            q = (slot - LAG) % NSLOT

            @pl.when(i >= LAG)
            def _():
                wait_rows_bytes(q, gsems[q])
                issue_scatter(q)

        for p in range(NSLOT):
            issue_idx(p, p)

        nmain = (nch // NSLOT) * NSLOT

        def step(m, carry):
            for p in range(NSLOT):
                chunk(NSLOT * m + p, p)
            return carry
        lax.fori_loop(0, nch // NSLOT, step, 0)
        for i in range(nmain, nch):
            chunk(jnp.int32(i), i % NSLOT)
        for i in range(nch - LAG, nch):
            q = i % NSLOT
            wait_rows_bytes(q, gsems[q])
            issue_scatter(q)
        for i in range(nch - NSLOT, nch):
            wait_scatter(i % NSLOT)
        plsc.subcore_barrier()

        nwb = ROWS_PER_TILE // WB

        def wbi(w, carry):
            r = row0 + w * WB
            pltpu.async_copy(acc_sh.at[pl.ds(r, WB), :],
                             out_hbm.at[c, pl.ds(r, WB), :], wsem)
            if with_counts:
                pltpu.async_copy(acc2_sh.at[pl.ds(r, WB), :],
                                 cnt_hbm.at[c, pl.ds(r, WB), :], wsem2)
            return carry
        lax.fori_loop(0, nwb, wbi, 0)

        def wbw(w, carry):
            pltpu.make_async_copy(
                acc_sh.at[pl.ds(row0, WB), :],
                out_hbm.at[c, pl.ds(row0, WB), :], wsem).wait()
            if with_counts:
                pltpu.make_async_copy(
                    acc2_sh.at[pl.ds(row0, WB), :],
                    cnt_hbm.at[c, pl.ds(row0, WB), :], wsem2).wait()
            return carry
        lax.fori_loop(0, nwb, wbw, 0)
    return body


_agg_body = _build_agg_body(False)
_agg_cnt_body = _build_agg_body(True)

_AGG_SCRATCH = [
    pltpu.VMEM((NSLOT, K), jnp.int32),
    pltpu.VMEM((NSLOT, K), jnp.int32),
    pltpu.VMEM((NSLOT, K), jnp.int32),
    pltpu.VMEM((NSLOT, K), jnp.int32),
    pltpu.VMEM((NSLOT, K), jnp.int32),
    pltpu.VMEM((NSLOT, K, HF), jnp.float32),
    pltpu.VMEM((WB, HF), jnp.float32),
]


def _sc_agg(src, dst, et, h_flat):
    return pl.kernel(
        _agg_body,
        out_type=jax.ShapeDtypeStruct((NCORES, 2 * NP, HF), jnp.float32),
        mesh=_mesh(),
        compiler_params=pltpu.CompilerParams(use_tc_tiling_on_sc=False),
        scratch_types=_AGG_SCRATCH + [
            pltpu.VMEM_SHARED((2 * NP, HF), jnp.float32),
        ] + [pltpu.SemaphoreType.DMA] * 13,
    )(src, dst, et, h_flat)


def _sc_agg_cnt(src, dst, et, h_flat):
    return pl.kernel(
        _agg_cnt_body,
        out_type=(
            jax.ShapeDtypeStruct((NCORES, 2 * NP, HF), jnp.float32),
            jax.ShapeDtypeStruct((NCORES, 2 * NP, 16), jnp.float32),
        ),
        mesh=_mesh(),
        compiler_params=pltpu.CompilerParams(use_tc_tiling_on_sc=False),
        scratch_types=_AGG_SCRATCH + [
            pltpu.VMEM((K, 16), jnp.float32),
            pltpu.VMEM_SHARED((2 * NP, HF), jnp.float32),
            pltpu.VMEM_SHARED((2 * NP, 16), jnp.float32),
        ] + [pltpu.SemaphoreType.DMA] * 18,
    )(src, dst, et, h_flat)


def _layer_tc_body(h_ref, a_ref, c_ref, root_ref, w_ref, b_ref, o_ref):
    h = jnp.concatenate([h_ref[0], h_ref[1]], axis=1)
    acc = jnp.dot(h, root_ref[...],
                  preferred_element_type=jnp.float32) + b_ref[...]
    for r in range(2):
        a = jnp.concatenate([a_ref[0, r], a_ref[1, r]], axis=1)
        inv = 1.0 / jnp.maximum(c_ref[r, :, 0:1], 1.0)
        acc = acc + jnp.dot(a * inv, w_ref[r],
                            preferred_element_type=jnp.float32)
    out = jnp.maximum(acc, 0.0)
    o_ref[0] = out[:, :HF]
    o_ref[1] = out[:, HF:]


def _tc_layer(h2, agg4, cnt, root, w, b2):
    return pl.pallas_call(
        _layer_tc_body,
        grid=(NGRID,),
        in_specs=[
            pl.BlockSpec((2, RBLK, HF), lambda i: (0, i, 0)),
            pl.BlockSpec((2, 2, RBLK, HF), lambda i: (0, 0, i, 0)),
            pl.BlockSpec((2, RBLK, 16), lambda i: (0, i, 0)),
            pl.BlockSpec((F, F), lambda i: (0, 0)),
            pl.BlockSpec((2, F, F), lambda i: (0, 0, 0)),
            pl.BlockSpec((1, F), lambda i: (0, 0)),
        ],
        out_specs=pl.BlockSpec((2, RBLK, HF), lambda i: (0, i, 0)),
        out_shape=jax.ShapeDtypeStruct((2, NP, HF), jnp.float32),
    )(h2, agg4, cnt, root, w, b2)


def _layer3_pool_body(h_ref, a_ref, c_ref, root_ref, w_ref, b_ref,
                      b3_ref, wl_ref, bl_ref, o_ref, s_acc, c_acc):
    i = pl.program_id(0)

    @pl.when(i == 0)
    def _():
        s_acc[...] = jnp.zeros_like(s_acc)
        c_acc[...] = jnp.zeros_like(c_acc)

    h = jnp.concatenate([h_ref[0], h_ref[1]], axis=1)
    acc = jnp.dot(h, root_ref[...],
                  preferred_element_type=jnp.float32) + b_ref[...]
    for r in range(2):
        a = jnp.concatenate([a_ref[0, r], a_ref[1, r]], axis=1)
        inv = 1.0 / jnp.maximum(c_ref[r, :, 0:1], 1.0)
        acc = acc + jnp.dot(a * inv, w_ref[r],
                            preferred_element_type=jnp.float32)
    out = jnp.maximum(acc, 0.0)

    bids = b3_ref[0]
    gids = lax.broadcasted_iota(jnp.int32, (NB, RBLK), 0)
    m = (gids == bids).astype(jnp.float32)
    s_acc[...] += jnp.dot(m, out, preferred_element_type=jnp.float32)
    c_acc[...] += jnp.sum(m, axis=1, keepdims=True)

    @pl.when(i == pl.num_programs(0) - 1)
    def _():
        g = s_acc[...] / jnp.maximum(c_acc[...], 1.0)
        o_ref[...] = jnp.dot(g, wl_ref[...],
                             preferred_element_type=jnp.float32) + bl_ref[...]


def _tc_layer3_pool(h2, agg4, cnt, root, w, b2, batch3, wl_pad, bl_pad):
    return pl.pallas_call(
        _layer3_pool_body,
        grid=(NGRID,),
        in_specs=[
            pl.BlockSpec((2, RBLK, HF), lambda i: (0, i, 0)),
            pl.BlockSpec((2, 2, RBLK, HF), lambda i: (0, 0, i, 0)),
            pl.BlockSpec((2, RBLK, 16), lambda i: (0, i, 0)),
            pl.BlockSpec((F, F), lambda i: (0, 0)),
            pl.BlockSpec((2, F, F), lambda i: (0, 0, 0)),
            pl.BlockSpec((1, F), lambda i: (0, 0)),
            pl.BlockSpec((1, 1, RBLK), lambda i: (i, 0, 0)),
            pl.BlockSpec((F, F), lambda i: (0, 0)),
            pl.BlockSpec((1, F), lambda i: (0, 0)),
        ],
        out_specs=pl.BlockSpec((NB, F), lambda i: (0, 0)),
        out_shape=jax.ShapeDtypeStruct((NB, F), jnp.float32),
        scratch_shapes=[
            pltpu.VMEM((NB, F), jnp.float32),
            pltpu.VMEM((NB, F), jnp.float32),
        ],
    )(h2, agg4, cnt, root, w, b2, batch3, wl_pad, bl_pad)


def kernel(x, edge_index, edge_attr, batch,
           W1, root1, b1, W2, root2, b2, W3, root3, b3, Wl, bl):
    src = edge_index[0].astype(jnp.int32)
    dst = edge_index[1].astype(jnp.int32)
    et = edge_attr.astype(jnp.int32)

    x_pad = jnp.zeros((NP, F), jnp.float32).at[:N].set(x)
    h_flat = jnp.concatenate([x_pad[:, :HF], x_pad[:, HF:]], axis=0)

    batch_p = jnp.concatenate(
        [batch.astype(jnp.int32), jnp.full((NP - N,), NB, jnp.int32)]
    ).reshape(NGRID, 1, RBLK)

    agg, counts = _sc_agg_cnt(src, dst, et, h_flat)
    cnt = counts[0].reshape(2, NP, 16)

    h2 = _tc_layer(h_flat.reshape(2, NP, HF), agg.reshape(2, 2, NP, HF),
                   cnt, root1, W1, b1.reshape(1, F))
    h_flat = h2.reshape(2 * NP, HF)

    agg = _sc_agg(src, dst, et, h_flat)
    h2 = _tc_layer(h_flat.reshape(2, NP, HF), agg.reshape(2, 2, NP, HF),
                   cnt, root2, W2, b2.reshape(1, F))
    h_flat = h2.reshape(2 * NP, HF)

    agg = _sc_agg(src, dst, et, h_flat)
    wl_pad = jnp.zeros((F, F), jnp.float32).at[:, :Wl.shape[1]].set(Wl)
    bl_pad = jnp.zeros((1, F), jnp.float32).at[0, :bl.shape[0]].set(bl)
    out = _tc_layer3_pool(h2, agg.reshape(2, 2, NP, HF), cnt,
                          root3, W3, b3.reshape(1, F),
                          batch_p, wl_pad, bl_pad)
    return out[:, :Wl.shape[1]]


# speedup vs baseline: 14.9242x; 1.0312x over previous
import jax
import jax.numpy as jnp
from jax import lax
from jax.experimental import pallas as pl
from jax.experimental.pallas import tpu as pltpu
from jax.experimental.pallas import tpu_sc as plsc

N = 10000
E = 320000
F = 128
HF = 64
NB = 64
RBLK = 512
NP = 10240
NGRID = NP // RBLK
NCORES = 2
NSUB = 16
K = 80
ROWS_PER_TILE = 2 * NP // NSUB
WB = 80
NSLOT_PLAIN, LAG_PLAIN = 8, 4
NSLOT_CNT, LAG_CNT = 4, 2


def _mesh():
    return plsc.VectorSubcoreMesh(
        core_axis_name="c", subcore_axis_name="s",
        num_cores=NCORES, num_subcores=NSUB)


def _zero16():
    return jnp.zeros((16,), jnp.float32)


def _build_agg_body(with_counts, NSLOT, LAG):
    def body(*args):
        if with_counts:
            (src_hbm, dst_hbm, et_hbm, h_hbm, out_hbm, cnt_hbm,
             src_v, dst_v, et_v, gi_v, si_v, rows_v, ones_v,
             acc_sh, acc2_sh, *sems) = args
        else:
            (src_hbm, dst_hbm, et_hbm, h_hbm, out_hbm,
             src_v, dst_v, et_v, gi_v, si_v, rows_v,
             acc_sh, *sems) = args
            cnt_hbm = ones_v = acc2_sh = None
        isems = sems[0:NSLOT]
        gsems = sems[NSLOT:2 * NSLOT]
        ssems = sems[2 * NSLOT:3 * NSLOT]
        if with_counts:
            osems = sems[3 * NSLOT:4 * NSLOT]
            wsem, wsem2 = sems[4 * NSLOT], sems[4 * NSLOT + 1]
        else:
            wsem = sems[3 * NSLOT]
        c = lax.axis_index("c")
        s = lax.axis_index("s")
        row0 = s * ROWS_PER_TILE

        def zrow(i, carry):
            for j in range(HF // 16):
                rows_v[0, i, pl.ds(j * 16, 16)] = _zero16()
            return carry
        lax.fori_loop(0, WB, zrow, 0)

        def zcp(w, carry):
            pltpu.sync_copy(rows_v.at[0],
                            acc_sh.at[pl.ds(row0 + w * WB, WB), :])
            return carry
        lax.fori_loop(0, ROWS_PER_TILE // WB, zcp, 0)

        if with_counts:
            def z2row(i, carry):
                ones_v[i, :] = _zero16()
                return carry
            lax.fori_loop(0, K, z2row, 0)

            def z2cp(w, carry):
                pltpu.sync_copy(ones_v.at[pl.ds(0, WB), :],
                                acc2_sh.at[pl.ds(row0 + w * WB, WB), :])
                return carry
            lax.fori_loop(0, ROWS_PER_TILE // WB, z2cp, 0)

            e0 = jnp.where(lax.iota(jnp.int32, 16) == 0,
                           jnp.float32(1.0), jnp.float32(0.0))

            def orow(i, carry):
                ones_v[i, :] = e0
                return carry
            lax.fori_loop(0, K, orow, 0)
        plsc.subcore_barrier()

        t_edges = E // NSUB
        base0 = s * t_edges
        goff = c * NP
        nch = t_edges // K

        def issue_idx(i, slot):
            b = base0 + i * K
            pltpu.async_copy(src_hbm.at[pl.ds(b, K)], src_v.at[slot],
                             isems[slot])
            pltpu.async_copy(dst_hbm.at[pl.ds(b, K)], dst_v.at[slot],
                             isems[slot])
            pltpu.async_copy(et_hbm.at[pl.ds(b, K)], et_v.at[slot],
                             isems[slot])

        def wait_idx(slot):
            for _ in range(3):
                pltpu.make_async_copy(
                    src_hbm.at[pl.ds(0, K)], src_v.at[slot],
                    isems[slot]).wait()

        def compute_idx(slot):
            for j in range(K // 16):
                sl = pl.ds(j * 16, 16)
                gi_v[slot, sl] = src_v[slot, sl] + goff
                si_v[slot, sl] = dst_v[slot, sl] + et_v[slot, sl] * NP

        def wait_rows_bytes(slot, sem):
            pltpu.make_async_copy(
                h_hbm.at[pl.ds(0, K)], rows_v.at[slot], sem).wait()

        def issue_scatter(slot):
            pltpu.async_copy(rows_v.at[slot], acc_sh.at[si_v.at[slot]],
                             ssems[slot], add=True)
            if with_counts:
                pltpu.async_copy(ones_v, acc2_sh.at[si_v.at[slot]],
                                 osems[slot], add=True)

        def wait_scatter(slot):
            wait_rows_bytes(slot, ssems[slot])
            if with_counts:
                pltpu.make_async_copy(
                    cnt_hbm.at[0, pl.ds(0, K), :], ones_v,
                    osems[slot]).wait()

        def chunk(i, slot):
            @pl.when(i >= NSLOT)
            def _():
                wait_scatter(slot)
            wait_idx(slot)
            compute_idx(slot)

            @pl.when(i + NSLOT < nch)
            def _():
                issue_idx(i + NSLOT, slot)
            pltpu.async_copy(h_hbm.at[gi_v.at[slot]], rows_v.at[slot],
                             gsems[slot])
            q = (slot - LAG) % NSLOT

            @pl.when(i >= LAG)
            def _():
                wait_rows_bytes(q, gsems[q])
                issue_scatter(q)

        for p in range(NSLOT):
            issue_idx(p, p)

        nmain = (nch // NSLOT) * NSLOT

        def step(m, carry):
            for p in range(NSLOT):
                chunk(NSLOT * m + p, p)
            return carry
        lax.fori_loop(0, nch // NSLOT, step, 0)
        for i in range(nmain, nch):
            chunk(jnp.int32(i), i % NSLOT)
        for i in range(nch - LAG, nch):
            q = i % NSLOT
            wait_rows_bytes(q, gsems[q])
            issue_scatter(q)
        for i in range(nch - NSLOT, nch):
            wait_scatter(i % NSLOT)
        plsc.subcore_barrier()

        nwb = ROWS_PER_TILE // WB

        def wbi(w, carry):
            r = row0 + w * WB
            pltpu.async_copy(acc_sh.at[pl.ds(r, WB), :],
                             out_hbm.at[c, pl.ds(r, WB), :], wsem)
            if with_counts:
                pltpu.async_copy(acc2_sh.at[pl.ds(r, WB), :],
                                 cnt_hbm.at[c, pl.ds(r, WB), :], wsem2)
            return carry
        lax.fori_loop(0, nwb, wbi, 0)

        def wbw(w, carry):
            pltpu.make_async_copy(
                acc_sh.at[pl.ds(row0, WB), :],
                out_hbm.at[c, pl.ds(row0, WB), :], wsem).wait()
            if with_counts:
                pltpu.make_async_copy(
                    acc2_sh.at[pl.ds(row0, WB), :],
                    cnt_hbm.at[c, pl.ds(row0, WB), :], wsem2).wait()
            return carry
        lax.fori_loop(0, nwb, wbw, 0)
    return body


_agg_body = _build_agg_body(False, NSLOT_PLAIN, LAG_PLAIN)
_agg_cnt_body = _build_agg_body(True, NSLOT_CNT, LAG_CNT)


def _agg_scratch(nslot):
    return [
        pltpu.VMEM((nslot, K), jnp.int32),
        pltpu.VMEM((nslot, K), jnp.int32),
        pltpu.VMEM((nslot, K), jnp.int32),
        pltpu.VMEM((nslot, K), jnp.int32),
        pltpu.VMEM((nslot, K), jnp.int32),
        pltpu.VMEM((nslot, K, HF), jnp.float32),
    ]


def _sc_agg(src, dst, et, h_flat):
    return pl.kernel(
        _agg_body,
        out_type=jax.ShapeDtypeStruct((NCORES, 2 * NP, HF), jnp.float32),
        mesh=_mesh(),
        compiler_params=pltpu.CompilerParams(use_tc_tiling_on_sc=False),
        scratch_types=_agg_scratch(NSLOT_PLAIN) + [
            pltpu.VMEM_SHARED((2 * NP, HF), jnp.float32),
        ] + [pltpu.SemaphoreType.DMA] * (3 * NSLOT_PLAIN + 1),
    )(src, dst, et, h_flat)


def _sc_agg_cnt(src, dst, et, h_flat):
    return pl.kernel(
        _agg_cnt_body,
        out_type=(
            jax.ShapeDtypeStruct((NCORES, 2 * NP, HF), jnp.float32),
            jax.ShapeDtypeStruct((NCORES, 2 * NP, 16), jnp.float32),
        ),
        mesh=_mesh(),
        compiler_params=pltpu.CompilerParams(use_tc_tiling_on_sc=False),
        scratch_types=_agg_scratch(NSLOT_CNT) + [
            pltpu.VMEM((K, 16), jnp.float32),
            pltpu.VMEM_SHARED((2 * NP, HF), jnp.float32),
            pltpu.VMEM_SHARED((2 * NP, 16), jnp.float32),
        ] + [pltpu.SemaphoreType.DMA] * (4 * NSLOT_CNT + 2),
    )(src, dst, et, h_flat)


def _layer_tc_body(h_ref, a_ref, c_ref, root_ref, w_ref, b_ref, o_ref):
    h = jnp.concatenate([h_ref[0], h_ref[1]], axis=1)
    acc = jnp.dot(h, root_ref[...],
                  preferred_element_type=jnp.float32) + b_ref[...]
    for r in range(2):
        a = jnp.concatenate([a_ref[0, r], a_ref[1, r]], axis=1)
        inv = 1.0 / jnp.maximum(c_ref[r, :, 0:1], 1.0)
        acc = acc + jnp.dot(a * inv, w_ref[r],
                            preferred_element_type=jnp.float32)
    out = jnp.maximum(acc, 0.0)
    o_ref[0] = out[:, :HF]
    o_ref[1] = out[:, HF:]


def _tc_layer(h2, agg4, cnt, root, w, b2):
    return pl.pallas_call(
        _layer_tc_body,
        grid=(NGRID,),
        in_specs=[
            pl.BlockSpec((2, RBLK, HF), lambda i: (0, i, 0)),
            pl.BlockSpec((2, 2, RBLK, HF), lambda i: (0, 0, i, 0)),
            pl.BlockSpec((2, RBLK, 16), lambda i: (0, i, 0)),
            pl.BlockSpec((F, F), lambda i: (0, 0)),
            pl.BlockSpec((2, F, F), lambda i: (0, 0, 0)),
            pl.BlockSpec((1, F), lambda i: (0, 0)),
        ],
        out_specs=pl.BlockSpec((2, RBLK, HF), lambda i: (0, i, 0)),
        out_shape=jax.ShapeDtypeStruct((2, NP, HF), jnp.float32),
    )(h2, agg4, cnt, root, w, b2)


def _layer3_pool_body(h_ref, a_ref, c_ref, root_ref, w_ref, b_ref,
                      b3_ref, wl_ref, bl_ref, o_ref, s_acc, c_acc):
    i = pl.program_id(0)

    @pl.when(i == 0)
    def _():
        s_acc[...] = jnp.zeros_like(s_acc)
        c_acc[...] = jnp.zeros_like(c_acc)

    h = jnp.concatenate([h_ref[0], h_ref[1]], axis=1)
    acc = jnp.dot(h, root_ref[...],
                  preferred_element_type=jnp.float32) + b_ref[...]
    for r in range(2):
        a = jnp.concatenate([a_ref[0, r], a_ref[1, r]], axis=1)
        inv = 1.0 / jnp.maximum(c_ref[r, :, 0:1], 1.0)
        acc = acc + jnp.dot(a * inv, w_ref[r],
                            preferred_element_type=jnp.float32)
    out = jnp.maximum(acc, 0.0)

    bids = b3_ref[0]
    gids = lax.broadcasted_iota(jnp.int32, (NB, RBLK), 0)
    m = (gids == bids).astype(jnp.float32)
    s_acc[...] += jnp.dot(m, out, preferred_element_type=jnp.float32)
    c_acc[...] += jnp.sum(m, axis=1, keepdims=True)

    @pl.when(i == pl.num_programs(0) - 1)
    def _():
        g = s_acc[...] / jnp.maximum(c_acc[...], 1.0)
        o_ref[...] = jnp.dot(g, wl_ref[...],
                             preferred_element_type=jnp.float32) + bl_ref[...]


def _tc_layer3_pool(h2, agg4, cnt, root, w, b2, batch3, wl_pad, bl_pad):
    return pl.pallas_call(
        _layer3_pool_body,
        grid=(NGRID,),
        in_specs=[
            pl.BlockSpec((2, RBLK, HF), lambda i: (0, i, 0)),
            pl.BlockSpec((2, 2, RBLK, HF), lambda i: (0, 0, i, 0)),
            pl.BlockSpec((2, RBLK, 16), lambda i: (0, i, 0)),
            pl.BlockSpec((F, F), lambda i: (0, 0)),
            pl.BlockSpec((2, F, F), lambda i: (0, 0, 0)),
            pl.BlockSpec((1, F), lambda i: (0, 0)),
            pl.BlockSpec((1, 1, RBLK), lambda i: (i, 0, 0)),
            pl.BlockSpec((F, F), lambda i: (0, 0)),
            pl.BlockSpec((1, F), lambda i: (0, 0)),
        ],
        out_specs=pl.BlockSpec((NB, F), lambda i: (0, 0)),
        out_shape=jax.ShapeDtypeStruct((NB, F), jnp.float32),
        scratch_shapes=[
            pltpu.VMEM((NB, F), jnp.float32),
            pltpu.VMEM((NB, F), jnp.float32),
        ],
    )(h2, agg4, cnt, root, w, b2, batch3, wl_pad, bl_pad)


def kernel(x, edge_index, edge_attr, batch,
           W1, root1, b1, W2, root2, b2, W3, root3, b3, Wl, bl):
    src = edge_index[0].astype(jnp.int32)
    dst = edge_index[1].astype(jnp.int32)
    et = edge_attr.astype(jnp.int32)

    x_pad = jnp.zeros((NP, F), jnp.float32).at[:N].set(x)
    h_flat = jnp.concatenate([x_pad[:, :HF], x_pad[:, HF:]], axis=0)

    batch_p = jnp.concatenate(
        [batch.astype(jnp.int32), jnp.full((NP - N,), NB, jnp.int32)]
    ).reshape(NGRID, 1, RBLK)

    agg, counts = _sc_agg_cnt(src, dst, et, h_flat)
    cnt = counts[0].reshape(2, NP, 16)

    h2 = _tc_layer(h_flat.reshape(2, NP, HF), agg.reshape(2, 2, NP, HF),
                   cnt, root1, W1, b1.reshape(1, F))
    h_flat = h2.reshape(2 * NP, HF)

    agg = _sc_agg(src, dst, et, h_flat)
    h2 = _tc_layer(h_flat.reshape(2, NP, HF), agg.reshape(2, 2, NP, HF),
                   cnt, root2, W2, b2.reshape(1, F))
    h_flat = h2.reshape(2 * NP, HF)

    agg = _sc_agg(src, dst, et, h_flat)
    wl_pad = jnp.zeros((F, F), jnp.float32).at[:, :Wl.shape[1]].set(Wl)
    bl_pad = jnp.zeros((1, F), jnp.float32).at[0, :bl.shape[0]].set(bl)
    out = _tc_layer3_pool(h2, agg.reshape(2, 2, NP, HF), cnt,
                          root3, W3, b3.reshape(1, F),
                          batch_p, wl_pad, bl_pad)
    return out[:, :Wl.shape[1]]


# speedup vs baseline: 15.6027x; 1.0455x over previous
import jax
import jax.numpy as jnp
from jax import lax
from jax.experimental import pallas as pl
from jax.experimental.pallas import tpu as pltpu
from jax.experimental.pallas import tpu_sc as plsc

N = 10000
E = 320000
F = 128
HF = 64
NB = 64
RBLK = 2048
NP = 10240
NGRID = NP // RBLK
NCORES = 2
NSUB = 16
K = 80
ROWS_PER_TILE = 2 * NP // NSUB
WB = 80
NSLOT_PLAIN, LAG_PLAIN = 8, 4
NSLOT_CNT, LAG_CNT = 4, 2


def _mesh():
    return plsc.VectorSubcoreMesh(
        core_axis_name="c", subcore_axis_name="s",
        num_cores=NCORES, num_subcores=NSUB)


def _zero16():
    return jnp.zeros((16,), jnp.float32)


def _build_agg_body(with_counts, NSLOT, LAG):
    def body(*args):
        if with_counts:
            (src_hbm, dst_hbm, et_hbm, h_hbm, out_hbm, cnt_hbm,
             src_v, dst_v, et_v, gi_v, si_v, rows_v, ones_v,
             acc_sh, acc2_sh, *sems) = args
        else:
            (src_hbm, dst_hbm, et_hbm, h_hbm, out_hbm,
             src_v, dst_v, et_v, gi_v, si_v, rows_v,
             acc_sh, *sems) = args
            cnt_hbm = ones_v = acc2_sh = None
        isems = sems[0:NSLOT]
        gsems = sems[NSLOT:2 * NSLOT]
        ssems = sems[2 * NSLOT:3 * NSLOT]
        if with_counts:
            osems = sems[3 * NSLOT:4 * NSLOT]
            wsem, wsem2 = sems[4 * NSLOT], sems[4 * NSLOT + 1]
        else:
            wsem = sems[3 * NSLOT]
        c = lax.axis_index("c")
        s = lax.axis_index("s")
        row0 = s * ROWS_PER_TILE

        def zrow(i, carry):
            for j in range(HF // 16):
                rows_v[0, i, pl.ds(j * 16, 16)] = _zero16()
            return carry
        lax.fori_loop(0, WB, zrow, 0)

        def zcp(w, carry):
            pltpu.sync_copy(rows_v.at[0],
                            acc_sh.at[pl.ds(row0 + w * WB, WB), :])
            return carry
        lax.fori_loop(0, ROWS_PER_TILE // WB, zcp, 0)

        if with_counts:
            def z2row(i, carry):
                ones_v[i, :] = _zero16()
                return carry
            lax.fori_loop(0, K, z2row, 0)

            def z2cp(w, carry):
                pltpu.sync_copy(ones_v.at[pl.ds(0, WB), :],
                                acc2_sh.at[pl.ds(row0 + w * WB, WB), :])
                return carry
            lax.fori_loop(0, ROWS_PER_TILE // WB, z2cp, 0)

            e0 = jnp.where(lax.iota(jnp.int32, 16) == 0,
                           jnp.float32(1.0), jnp.float32(0.0))

            def orow(i, carry):
                ones_v[i, :] = e0
                return carry
            lax.fori_loop(0, K, orow, 0)
        plsc.subcore_barrier()

        t_edges = E // NSUB
        base0 = s * t_edges
        goff = c * NP
        nch = t_edges // K

        def issue_idx(i, slot):
            b = base0 + i * K
            pltpu.async_copy(src_hbm.at[pl.ds(b, K)], src_v.at[slot],
                             isems[slot])
            pltpu.async_copy(dst_hbm.at[pl.ds(b, K)], dst_v.at[slot],
                             isems[slot])
            pltpu.async_copy(et_hbm.at[pl.ds(b, K)], et_v.at[slot],
                             isems[slot])

        def wait_idx(slot):
            for _ in range(3):
                pltpu.make_async_copy(
                    src_hbm.at[pl.ds(0, K)], src_v.at[slot],
                    isems[slot]).wait()

        def compute_idx(slot):
            for j in range(K // 16):
                sl = pl.ds(j * 16, 16)
                gi_v[slot, sl] = src_v[slot, sl] + goff
                si_v[slot, sl] = dst_v[slot, sl] + et_v[slot, sl] * NP

        def wait_rows_bytes(slot, sem):
            pltpu.make_async_copy(
                h_hbm.at[pl.ds(0, K)], rows_v.at[slot], sem).wait()

        def issue_scatter(slot):
            pltpu.async_copy(rows_v.at[slot], acc_sh.at[si_v.at[slot]],
                             ssems[slot], add=True)
            if with_counts:
                pltpu.async_copy(ones_v, acc2_sh.at[si_v.at[slot]],
                                 osems[slot], add=True)

        def wait_scatter(slot):
            wait_rows_bytes(slot, ssems[slot])
            if with_counts:
                pltpu.make_async_copy(
                    cnt_hbm.at[0, pl.ds(0, K), :], ones_v,
                    osems[slot]).wait()

        def chunk(i, slot):
            @pl.when(i >= NSLOT)
            def _():
                wait_scatter(slot)
            wait_idx(slot)
            compute_idx(slot)

            @pl.when(i + NSLOT < nch)
            def _():
                issue_idx(i + NSLOT, slot)
            pltpu.async_copy(h_hbm.at[gi_v.at[slot]], rows_v.at[slot],
                             gsems[slot])
            q = (slot - LAG) % NSLOT

            @pl.when(i >= LAG)
            def _():
                wait_rows_bytes(q, gsems[q])
                issue_scatter(q)

        for p in range(NSLOT):
            issue_idx(p, p)

        nmain = (nch // NSLOT) * NSLOT

        def step(m, carry):
            for p in range(NSLOT):
                chunk(NSLOT * m + p, p)
            return carry
        lax.fori_loop(0, nch // NSLOT, step, 0)
        for i in range(nmain, nch):
            chunk(jnp.int32(i), i % NSLOT)
        for i in range(nch - LAG, nch):
            q = i % NSLOT
            wait_rows_bytes(q, gsems[q])
            issue_scatter(q)
        for i in range(nch - NSLOT, nch):
            wait_scatter(i % NSLOT)
        plsc.subcore_barrier()

        nwb = ROWS_PER_TILE // WB

        def wbi(w, carry):
            r = row0 + w * WB
            pltpu.async_copy(acc_sh.at[pl.ds(r, WB), :],
                             out_hbm.at[c, pl.ds(r, WB), :], wsem)
            if with_counts:
                pltpu.async_copy(acc2_sh.at[pl.ds(r, WB), :],
                                 cnt_hbm.at[c, pl.ds(r, WB), :], wsem2)
            return carry
        lax.fori_loop(0, nwb, wbi, 0)

        def wbw(w, carry):
            pltpu.make_async_copy(
                acc_sh.at[pl.ds(row0, WB), :],
                out_hbm.at[c, pl.ds(row0, WB), :], wsem).wait()
            if with_counts:
                pltpu.make_async_copy(
                    acc2_sh.at[pl.ds(row0, WB), :],
                    cnt_hbm.at[c, pl.ds(row0, WB), :], wsem2).wait()
            return carry
        lax.fori_loop(0, nwb, wbw, 0)
    return body


_agg_body = _build_agg_body(False, NSLOT_PLAIN, LAG_PLAIN)
_agg_cnt_body = _build_agg_body(True, NSLOT_CNT, LAG_CNT)


def _agg_scratch(nslot):
    return [
        pltpu.VMEM((nslot, K), jnp.int32),
        pltpu.VMEM((nslot, K), jnp.int32),
        pltpu.VMEM((nslot, K), jnp.int32),
        pltpu.VMEM((nslot, K), jnp.int32),
        pltpu.VMEM((nslot, K), jnp.int32),
        pltpu.VMEM((nslot, K, HF), jnp.float32),
    ]


def _sc_agg(src, dst, et, h_flat):
    return pl.kernel(
        _agg_body,
        out_type=jax.ShapeDtypeStruct((NCORES, 2 * NP, HF), jnp.float32),
        mesh=_mesh(),
        compiler_params=pltpu.CompilerParams(use_tc_tiling_on_sc=False),
        scratch_types=_agg_scratch(NSLOT_PLAIN) + [
            pltpu.VMEM_SHARED((2 * NP, HF), jnp.float32),
        ] + [pltpu.SemaphoreType.DMA] * (3 * NSLOT_PLAIN + 1),
    )(src, dst, et, h_flat)


def _sc_agg_cnt(src, dst, et, h_flat):
    return pl.kernel(
        _agg_cnt_body,
        out_type=(
            jax.ShapeDtypeStruct((NCORES, 2 * NP, HF), jnp.float32),
            jax.ShapeDtypeStruct((NCORES, 2 * NP, 16), jnp.float32),
        ),
        mesh=_mesh(),
        compiler_params=pltpu.CompilerParams(use_tc_tiling_on_sc=False),
        scratch_types=_agg_scratch(NSLOT_CNT) + [
            pltpu.VMEM((K, 16), jnp.float32),
            pltpu.VMEM_SHARED((2 * NP, HF), jnp.float32),
            pltpu.VMEM_SHARED((2 * NP, 16), jnp.float32),
        ] + [pltpu.SemaphoreType.DMA] * (4 * NSLOT_CNT + 2),
    )(src, dst, et, h_flat)


def _layer_tc_body(h_ref, a_ref, c_ref, root_ref, w_ref, b_ref, o_ref):
    h = jnp.concatenate([h_ref[0], h_ref[1]], axis=1)
    acc = jnp.dot(h, root_ref[...],
                  preferred_element_type=jnp.float32) + b_ref[...]
    for r in range(2):
        a = jnp.concatenate([a_ref[0, r], a_ref[1, r]], axis=1)
        inv = 1.0 / jnp.maximum(c_ref[r, :, 0:1], 1.0)
        acc = acc + jnp.dot(a * inv, w_ref[r],
                            preferred_element_type=jnp.float32)
    out = jnp.maximum(acc, 0.0)
    o_ref[0] = out[:, :HF]
    o_ref[1] = out[:, HF:]


def _tc_layer(h2, agg4, cnt, root, w, b2):
    return pl.pallas_call(
        _layer_tc_body,
        grid=(NGRID,),
        in_specs=[
            pl.BlockSpec((2, RBLK, HF), lambda i: (0, i, 0)),
            pl.BlockSpec((2, 2, RBLK, HF), lambda i: (0, 0, i, 0)),
            pl.BlockSpec((2, RBLK, 16), lambda i: (0, i, 0)),
            pl.BlockSpec((F, F), lambda i: (0, 0)),
            pl.BlockSpec((2, F, F), lambda i: (0, 0, 0)),
            pl.BlockSpec((1, F), lambda i: (0, 0)),
        ],
        out_specs=pl.BlockSpec((2, RBLK, HF), lambda i: (0, i, 0)),
        out_shape=jax.ShapeDtypeStruct((2, NP, HF), jnp.float32),
    )(h2, agg4, cnt, root, w, b2)


def _layer3_pool_body(h_ref, a_ref, c_ref, root_ref, w_ref, b_ref,
                      b3_ref, wl_ref, bl_ref, o_ref, s_acc, c_acc):
    i = pl.program_id(0)

    @pl.when(i == 0)
    def _():
        s_acc[...] = jnp.zeros_like(s_acc)
        c_acc[...] = jnp.zeros_like(c_acc)

    h = jnp.concatenate([h_ref[0], h_ref[1]], axis=1)
    acc = jnp.dot(h, root_ref[...],
                  preferred_element_type=jnp.float32) + b_ref[...]
    for r in range(2):
        a = jnp.concatenate([a_ref[0, r], a_ref[1, r]], axis=1)
        inv = 1.0 / jnp.maximum(c_ref[r, :, 0:1], 1.0)
        acc = acc + jnp.dot(a * inv, w_ref[r],
                            preferred_element_type=jnp.float32)
    out = jnp.maximum(acc, 0.0)

    bids = b3_ref[0]
    gids = lax.broadcasted_iota(jnp.int32, (NB, RBLK), 0)
    m = (gids == bids).astype(jnp.float32)
    s_acc[...] += jnp.dot(m, out, preferred_element_type=jnp.float32)
    c_acc[...] += jnp.sum(m, axis=1, keepdims=True)

    @pl.when(i == pl.num_programs(0) - 1)
    def _():
        g = s_acc[...] / jnp.maximum(c_acc[...], 1.0)
        o_ref[...] = jnp.dot(g, wl_ref[...],
                             preferred_element_type=jnp.float32) + bl_ref[...]


def _tc_layer3_pool(h2, agg4, cnt, root, w, b2, batch3, wl_pad, bl_pad):
    return pl.pallas_call(
        _layer3_pool_body,
        grid=(NGRID,),
        in_specs=[
            pl.BlockSpec((2, RBLK, HF), lambda i: (0, i, 0)),
            pl.BlockSpec((2, 2, RBLK, HF), lambda i: (0, 0, i, 0)),
            pl.BlockSpec((2, RBLK, 16), lambda i: (0, i, 0)),
            pl.BlockSpec((F, F), lambda i: (0, 0)),
            pl.BlockSpec((2, F, F), lambda i: (0, 0, 0)),
            pl.BlockSpec((1, F), lambda i: (0, 0)),
            pl.BlockSpec((1, 1, RBLK), lambda i: (i, 0, 0)),
            pl.BlockSpec((F, F), lambda i: (0, 0)),
            pl.BlockSpec((1, F), lambda i: (0, 0)),
        ],
        out_specs=pl.BlockSpec((NB, F), lambda i: (0, 0)),
        out_shape=jax.ShapeDtypeStruct((NB, F), jnp.float32),
        scratch_shapes=[
            pltpu.VMEM((NB, F), jnp.float32),
            pltpu.VMEM((NB, F), jnp.float32),
        ],
    )(h2, agg4, cnt, root, w, b2, batch3, wl_pad, bl_pad)


def kernel(x, edge_index, edge_attr, batch,
           W1, root1, b1, W2, root2, b2, W3, root3, b3, Wl, bl):
    src = edge_index[0].astype(jnp.int32)
    dst = edge_index[1].astype(jnp.int32)
    et = edge_attr.astype(jnp.int32)

    x_pad = jnp.zeros((NP, F), jnp.float32).at[:N].set(x)
    h_flat = jnp.concatenate([x_pad[:, :HF], x_pad[:, HF:]], axis=0)

    batch_p = jnp.concatenate(
        [batch.astype(jnp.int32), jnp.full((NP - N,), NB, jnp.int32)]
    ).reshape(NGRID, 1, RBLK)

    agg, counts = _sc_agg_cnt(src, dst, et, h_flat)
    cnt = counts[0].reshape(2, NP, 16)

    h2 = _tc_layer(h_flat.reshape(2, NP, HF), agg.reshape(2, 2, NP, HF),
                   cnt, root1, W1, b1.reshape(1, F))
    h_flat = h2.reshape(2 * NP, HF)

    agg = _sc_agg(src, dst, et, h_flat)
    h2 = _tc_layer(h_flat.reshape(2, NP, HF), agg.reshape(2, 2, NP, HF),
                   cnt, root2, W2, b2.reshape(1, F))
    h_flat = h2.reshape(2 * NP, HF)

    agg = _sc_agg(src, dst, et, h_flat)
    wl_pad = jnp.zeros((F, F), jnp.float32).at[:, :Wl.shape[1]].set(Wl)
    bl_pad = jnp.zeros((1, F), jnp.float32).at[0, :bl.shape[0]].set(bl)
    out = _tc_layer3_pool(h2, agg.reshape(2, 2, NP, HF), cnt,
                          root3, W3, b3.reshape(1, F),
                          batch_p, wl_pad, bl_pad)
    return out[:, :Wl.shape[1]]
